# Initial kernel scaffold; baseline (speedup 1.0000x reference)
import jax
import jax.numpy as jnp
from jax.experimental import pallas as pl


def kernel(embs, scores):
    raise NotImplementedError("write your pallas kernel here")


# speedup vs baseline: 5.4400x; 5.4400x over previous
import functools

import jax
import jax.numpy as jnp
from jax import lax
from jax.experimental import pallas as pl
from jax.experimental.pallas import tpu as pltpu
from jax.experimental.pallas import tpu_sc as plsc

B = 8
L0 = 8192
E = 128
L1, L2, L3 = 4096, 2048, 1024

NC, NS, LANES = 2, 16, 16
NW = NC * NS
ROWS_PER_W = (B * L3) // NW
GROUP = 16
NGROUPS = ROWS_PER_W // GROUP


def _bitonic_desc(s, pos, iota):
    length = s.shape[1]
    k = 2
    while k <= length:
        j = k // 2
        while j >= 1:
            first = (iota & j) == 0
            sp = jnp.where(first, jnp.roll(s, -j, axis=1), jnp.roll(s, j, axis=1))
            pp = jnp.where(first, jnp.roll(pos, -j, axis=1), jnp.roll(pos, j, axis=1))
            beats = (sp > s) | ((sp == s) & (pp < pos))
            asc = (iota & k) != 0
            take = beats ^ (~first) ^ asc
            s = jnp.where(take, sp, s)
            pos = jnp.where(take, pp, pos)
            j //= 2
        k *= 2
    return s, pos


def _sort_body(s_ref, ss_ref, perm_ref):
    s = s_ref[:, :]
    iota = lax.broadcasted_iota(jnp.int32, s.shape, 1)
    ss, perm = _bitonic_desc(s, iota, iota)
    ss_ref[:, :] = ss
    perm_ref[:, :] = perm


def _tc_sort_layer(s):
    length = s.shape[1]
    return pl.pallas_call(
        _sort_body,
        out_shape=[
            jax.ShapeDtypeStruct((B, length), jnp.float32),
            jax.ShapeDtypeStruct((B, length), jnp.int32),
        ],
    )(s)


def _pair_combine(ss):
    half = ss.shape[1] // 2
    sc = jnp.stack([ss[:, :half], ss[:, half:][:, ::-1]], axis=1)
    exped = jnp.power(2.0, sc)
    w = jax.nn.softmax(exped, axis=1)
    s_new = (sc * w).sum(axis=1)
    return w[:, 0, :], s_new


def _splat(v):
    return jnp.zeros((LANES,), jnp.int32) + v


def _sc_combine(table, p0, w0, p1, w1, p2, w2):
    mesh = plsc.VectorSubcoreMesh(core_axis_name="c", subcore_axis_name="s")

    @functools.partial(
        pl.kernel,
        mesh=mesh,
        compiler_params=pltpu.CompilerParams(needs_layout_passes=False),
        out_type=jax.ShapeDtypeStruct((B * L3, E), jnp.float32),
        scratch_types=[
            pltpu.VMEM((L0,), jnp.int32),
            pltpu.VMEM((L1,), jnp.float32),
            pltpu.VMEM((L1,), jnp.int32),
            pltpu.VMEM((L2,), jnp.float32),
            pltpu.VMEM((L2,), jnp.int32),
            pltpu.VMEM((L3,), jnp.float32),
            pltpu.VMEM((GROUP * 8,), jnp.int32),
            pltpu.VMEM((GROUP * 8,), jnp.float32),
            pltpu.VMEM((GROUP * 8, E), jnp.float32),
            pltpu.VMEM((GROUP, E), jnp.float32),
            pltpu.SemaphoreType.DMA,
        ],
    )
    def k(table_hbm, p0_hbm, w0_hbm, p1_hbm, w1_hbm, p2_hbm, w2_hbm, out_hbm,
          p0_v, w0_v, p1_v, w1_v, p2_v, w2_v, idx_v, wt_v, rows_v, ob_v, sem):
        wid = lax.axis_index("s") * NC + lax.axis_index("c")
        b = wid // (NW // B)
        q = wid % (NW // B)
        pltpu.sync_copy(p0_hbm.at[b], p0_v)
        pltpu.sync_copy(w0_hbm.at[b], w0_v)
        pltpu.sync_copy(p1_hbm.at[b], p1_v)
        pltpu.sync_copy(w1_hbm.at[b], w1_v)
        pltpu.sync_copy(p2_hbm.at[b], p2_v)
        pltpu.sync_copy(w2_hbm.at[b], w2_v)
        iota = lax.iota(jnp.int32, LANES)

        def group(g, carry):
            jv = q * ROWS_PER_W + g * GROUP + iota
            a2 = plsc.load_gather(p2_v, [jv])
            b2 = plsc.load_gather(p2_v, [(L2 - 1) - jv])
            w2v = plsc.load_gather(w2_v, [jv])
            lvl1 = []
            for p, w in ((a2, w2v), (b2, 1.0 - w2v)):
                pa = plsc.load_gather(p1_v, [p])
                pb = plsc.load_gather(p1_v, [(L1 - 1) - p])
                w1v = plsc.load_gather(w1_v, [p])
                lvl1.append((pa, w * w1v))
                lvl1.append((pb, w * (1.0 - w1v)))
            kk = 0
            for p, w in lvl1:
                ia = plsc.load_gather(p0_v, [p])
                ib = plsc.load_gather(p0_v, [(L0 - 1) - p])
                w0v = plsc.load_gather(w0_v, [p])
                for idx, wt in ((ia, w * w0v), (ib, w * (1.0 - w0v))):
                    pos = iota * 8 + kk
                    plsc.store_scatter(idx_v, [pos], idx + b * L0)
                    plsc.store_scatter(wt_v, [pos], wt)
                    kk += 1
            pltpu.async_copy(table_hbm.at[idx_v], rows_v, sem).wait()

            def inner(jj, c2):
                accs = [jnp.zeros((LANES,), jnp.float32)] * 8
                for t in range(8):
                    rs = _splat(jj * 8 + t)
                    wv = plsc.load_gather(wt_v, [rs])
                    for d in range(8):
                        accs[d] = accs[d] + wv * plsc.load_gather(
                            rows_v, [rs, d * LANES + iota])
                js = _splat(jj)
                for d in range(8):
                    plsc.store_scatter(ob_v, [js, d * LANES + iota], accs[d])
                return c2

            lax.fori_loop(0, GROUP, inner, 0)
            row0 = b * L3 + q * ROWS_PER_W + g * GROUP
            pltpu.sync_copy(ob_v, out_hbm.at[pl.ds(row0, GROUP)])
            return carry

        lax.fori_loop(0, NGROUPS, group, 0)

    return k(table, p0, w0, p1, w1, p2, w2)


def kernel(embs, scores):
    s = scores[..., 0]
    ss0, p0 = _tc_sort_layer(s)
    w0, s1 = _pair_combine(ss0)
    ss1, p1 = _tc_sort_layer(s1)
    w1, s2 = _pair_combine(ss1)
    ss2, p2 = _tc_sort_layer(s2)
    w2, s3 = _pair_combine(ss2)
    table = embs.reshape(B * L0, E)
    out = _sc_combine(table, p0, w0, p1, w1, p2, w2)
    return out.reshape(B, L3, E), s3


# speedup vs baseline: 7.5726x; 1.3920x over previous
import functools

import jax
import jax.numpy as jnp
from jax import lax
from jax.experimental import pallas as pl
from jax.experimental.pallas import tpu as pltpu
from jax.experimental.pallas import tpu_sc as plsc

B = 8
L0 = 8192
E = 128
L1, L2, L3 = 4096, 2048, 1024

NC, NS, LANES = 2, 16, 16
NW = NC * NS
ROWS_PER_W = (B * L3) // NW
GROUP = 16
NGROUPS = ROWS_PER_W // GROUP


def _bitonic_desc(s, pos, iota):
    length = s.shape[1]
    k = 2
    while k <= length:
        j = k // 2
        while j >= 1:
            first = (iota & j) == 0
            sp = jnp.where(first, jnp.roll(s, -j, axis=1), jnp.roll(s, j, axis=1))
            pp = jnp.where(first, jnp.roll(pos, -j, axis=1), jnp.roll(pos, j, axis=1))
            beats = (sp > s) | ((sp == s) & (pp < pos))
            asc = (iota & k) != 0
            take = beats ^ (~first) ^ asc
            s = jnp.where(take, sp, s)
            pos = jnp.where(take, pp, pos)
            j //= 2
        k *= 2
    return s, pos


def _flip_lanes(x):
    length = x.shape[1]
    iota = lax.broadcasted_iota(jnp.int32, x.shape, 1)
    j = 1
    while j < length:
        first = (iota & j) == 0
        x = jnp.where(first, jnp.roll(x, -j, axis=1), jnp.roll(x, j, axis=1))
        j *= 2
    return x


def _pair_combine(ss):
    half = ss.shape[1] // 2
    st = ss[:, :half]
    sb = _flip_lanes(ss[:, half:])
    xl = jnp.power(2.0, st)
    xr = jnp.power(2.0, sb)
    m = jnp.maximum(xl, xr)
    el = jnp.exp(xl - m)
    er = jnp.exp(xr - m)
    den = el + er
    wl = el / den
    wr = er / den
    s_new = st * wl + sb * wr
    return wl, s_new


def _layer_body(s_ref, perm_ref, wl_ref, snew_ref):
    s = s_ref[:, :]
    iota = lax.broadcasted_iota(jnp.int32, s.shape, 1)
    ss, perm = _bitonic_desc(s, iota, iota)
    perm_ref[:, :] = perm
    wl, s_new = _pair_combine(ss)
    wl_ref[:, :] = wl
    snew_ref[:, :] = s_new


def _tc_layer(s):
    length = s.shape[1]
    return pl.pallas_call(
        _layer_body,
        out_shape=[
            jax.ShapeDtypeStruct((B, length), jnp.int32),
            jax.ShapeDtypeStruct((B, length // 2), jnp.float32),
            jax.ShapeDtypeStruct((B, length // 2), jnp.float32),
        ],
    )(s)


def _splat(v):
    return jnp.zeros((LANES,), jnp.int32) + v


def _sc_combine(table, p0, w0, p1, w1, p2, w2):
    mesh = plsc.VectorSubcoreMesh(core_axis_name="c", subcore_axis_name="s")

    @functools.partial(
        pl.kernel,
        mesh=mesh,
        compiler_params=pltpu.CompilerParams(needs_layout_passes=False),
        out_type=jax.ShapeDtypeStruct((B * L3, E), jnp.float32),
        scratch_types=[
            pltpu.VMEM((L0,), jnp.int32),
            pltpu.VMEM((L1,), jnp.float32),
            pltpu.VMEM((L1,), jnp.int32),
            pltpu.VMEM((L2,), jnp.float32),
            pltpu.VMEM((L2,), jnp.int32),
            pltpu.VMEM((L3,), jnp.float32),
            pltpu.VMEM((ROWS_PER_W * 8,), jnp.int32),
            pltpu.VMEM((ROWS_PER_W * 8,), jnp.float32),
            pltpu.VMEM((GROUP * 8, E), jnp.float32),
            pltpu.VMEM((GROUP * 8, E), jnp.float32),
            pltpu.VMEM((GROUP, E), jnp.float32),
            pltpu.SemaphoreType.DMA,
            pltpu.SemaphoreType.DMA,
        ],
    )
    def k(table_hbm, p0_hbm, w0_hbm, p1_hbm, w1_hbm, p2_hbm, w2_hbm, out_hbm,
          p0_v, w0_v, p1_v, w1_v, p2_v, w2_v, idx_v, wt_v, rows_a, rows_b,
          ob_v, sem_a, sem_b):
        wid = lax.axis_index("s") * NC + lax.axis_index("c")
        b = wid // (NW // B)
        q = wid % (NW // B)
        pltpu.sync_copy(p0_hbm.at[b], p0_v)
        pltpu.sync_copy(w0_hbm.at[b], w0_v)
        pltpu.sync_copy(p1_hbm.at[b], p1_v)
        pltpu.sync_copy(w1_hbm.at[b], w1_v)
        pltpu.sync_copy(p2_hbm.at[b], p2_v)
        pltpu.sync_copy(w2_hbm.at[b], w2_v)
        iota = lax.iota(jnp.int32, LANES)
        bufs = ((rows_a, sem_a), (rows_b, sem_b))

        def compose(g, carry):
            jv = q * ROWS_PER_W + g * GROUP + iota
            a2 = plsc.load_gather(p2_v, [jv])
            b2 = plsc.load_gather(p2_v, [(L2 - 1) - jv])
            w2v = plsc.load_gather(w2_v, [jv])
            lvl1 = []
            for p, w in ((a2, w2v), (b2, 1.0 - w2v)):
                pa = plsc.load_gather(p1_v, [p])
                pb = plsc.load_gather(p1_v, [(L1 - 1) - p])
                w1v = plsc.load_gather(w1_v, [p])
                lvl1.append((pa, w * w1v))
                lvl1.append((pb, w * (1.0 - w1v)))
            kk = 0
            base = g * (GROUP * 8)
            for p, w in lvl1:
                ia = plsc.load_gather(p0_v, [p])
                ib = plsc.load_gather(p0_v, [(L0 - 1) - p])
                w0v = plsc.load_gather(w0_v, [p])
                for idx, wt in ((ia, w * w0v), (ib, w * (1.0 - w0v))):
                    pos = base + iota * 8 + kk
                    plsc.store_scatter(idx_v, [pos], idx + b * L0)
                    plsc.store_scatter(wt_v, [pos], wt)
                    kk += 1
            return carry

        lax.fori_loop(0, NGROUPS, compose, 0)

        def gather_of(g, rows_v, sem):
            return pltpu.make_async_copy(
                table_hbm.at[idx_v.at[pl.ds(g * (GROUP * 8), GROUP * 8)]],
                rows_v, sem)

        gather_of(0, rows_a, sem_a).start()
        gather_of(1, rows_b, sem_b).start()

        def outer(i, carry):
            for slot in range(2):
                g = i * 2 + slot
                rows_v, sem = bufs[slot]
                gather_of(g, rows_v, sem).wait()
                wbase = g * (GROUP * 8)

                def inner(jj, c2):
                    accs = [jnp.zeros((LANES,), jnp.float32)] * 8
                    for t in range(8):
                        rs = _splat(jj * 8 + t)
                        wv = plsc.load_gather(wt_v, [wbase + rs])
                        for d in range(8):
                            accs[d] = accs[d] + wv * plsc.load_gather(
                                rows_v, [rs, d * LANES + iota])
                    js = _splat(jj)
                    for d in range(8):
                        plsc.store_scatter(ob_v, [js, d * LANES + iota],
                                           accs[d])
                    return c2

                lax.fori_loop(0, GROUP, inner, 0)
                row0 = b * L3 + q * ROWS_PER_W + g * GROUP
                pltpu.sync_copy(ob_v, out_hbm.at[pl.ds(row0, GROUP)])

                @pl.when(g + 2 < NGROUPS)
                def _():
                    gather_of(g + 2, rows_v, sem).start()
            return carry

        lax.fori_loop(0, NGROUPS // 2, outer, 0)

    return k(table, p0, w0, p1, w1, p2, w2)


def kernel(embs, scores):
    s = scores[..., 0]
    p0, w0, s1 = _tc_layer(s)
    p1, w1, s2 = _tc_layer(s1)
    p2, w2, s3 = _tc_layer(s2)
    table = embs.reshape(B * L0, E)
    out = _sc_combine(table, p0, w0, p1, w1, p2, w2)
    return out.reshape(B, L3, E), s3


# speedup vs baseline: 7.6263x; 1.0071x over previous
import functools

import jax
import jax.numpy as jnp
from jax import lax
from jax.experimental import pallas as pl
from jax.experimental.pallas import tpu as pltpu
from jax.experimental.pallas import tpu_sc as plsc

B = 8
L0 = 8192
E = 128
L1, L2, L3 = 4096, 2048, 1024

NC, NS, LANES = 2, 16, 16
NW = NC * NS
ROWS_PER_W = (B * L3) // NW
GROUP = 16
NGROUPS = ROWS_PER_W // GROUP


def _bitonic_desc(s, pos, iota):
    length = s.shape[1]
    k = 2
    while k <= length:
        j = k // 2
        while j >= 1:
            first = (iota & j) == 0
            sp = jnp.where(first, jnp.roll(s, -j, axis=1), jnp.roll(s, j, axis=1))
            pp = jnp.where(first, jnp.roll(pos, -j, axis=1), jnp.roll(pos, j, axis=1))
            beats = (sp > s) | ((sp == s) & (pp < pos))
            asc = (iota & k) != 0
            take = beats ^ (~first) ^ asc
            s = jnp.where(take, sp, s)
            pos = jnp.where(take, pp, pos)
            j //= 2
        k *= 2
    return s, pos


def _flip_lanes(x):
    length = x.shape[1]
    iota = lax.broadcasted_iota(jnp.int32, x.shape, 1)
    j = 1
    while j < length:
        first = (iota & j) == 0
        x = jnp.where(first, jnp.roll(x, -j, axis=1), jnp.roll(x, j, axis=1))
        j *= 2
    return x


def _pair_combine(ss):
    half = ss.shape[1] // 2
    st = ss[:, :half]
    sb = _flip_lanes(ss[:, half:])
    xl = jnp.power(2.0, st)
    xr = jnp.power(2.0, sb)
    m = jnp.maximum(xl, xr)
    el = jnp.exp(xl - m)
    er = jnp.exp(xr - m)
    den = el + er
    wl = el / den
    wr = er / den
    s_new = st * wl + sb * wr
    return wl, s_new


def _layer_body(s_ref, perm_ref, wl_ref, snew_ref):
    s = s_ref[:, :]
    iota = lax.broadcasted_iota(jnp.int32, s.shape, 1)
    ss, perm = _bitonic_desc(s, iota, iota)
    perm_ref[:, :] = perm
    wl, s_new = _pair_combine(ss)
    wl_ref[:, :] = wl
    snew_ref[:, :] = s_new


def _tc_layer(s):
    length = s.shape[1]
    return pl.pallas_call(
        _layer_body,
        out_shape=[
            jax.ShapeDtypeStruct((B, length), jnp.int32),
            jax.ShapeDtypeStruct((B, length // 2), jnp.float32),
            jax.ShapeDtypeStruct((B, length // 2), jnp.float32),
        ],
    )(s)


def _layer12_body(s_ref, p1_ref, w1_ref, p2_ref, w2_ref, os_ref):
    s = s_ref[:, :]
    iota = lax.broadcasted_iota(jnp.int32, (B, L1), 1)
    ss1, p1 = _bitonic_desc(s, iota, iota)
    p1_ref[:, :] = p1
    w1, s2 = _pair_combine(ss1)
    w1_ref[:, :] = w1
    ss2, p2 = _bitonic_desc(s2, iota[:, :L2], iota[:, :L2])
    p2_ref[:, :] = p2
    w2, s3 = _pair_combine(ss2)
    w2_ref[:, :] = w2
    os_ref[:, :] = s3


def _tc_layers12(s1):
    return pl.pallas_call(
        _layer12_body,
        out_shape=[
            jax.ShapeDtypeStruct((B, L1), jnp.int32),
            jax.ShapeDtypeStruct((B, L2), jnp.float32),
            jax.ShapeDtypeStruct((B, L2), jnp.int32),
            jax.ShapeDtypeStruct((B, L3), jnp.float32),
            jax.ShapeDtypeStruct((B, L3), jnp.float32),
        ],
    )(s1)


def _splat(v):
    return jnp.zeros((LANES,), jnp.int32) + v


def _sc_combine(table, p0, w0, p1, w1, p2, w2):
    mesh = plsc.VectorSubcoreMesh(core_axis_name="c", subcore_axis_name="s")

    @functools.partial(
        pl.kernel,
        mesh=mesh,
        compiler_params=pltpu.CompilerParams(needs_layout_passes=False),
        out_type=jax.ShapeDtypeStruct((B * L3, E), jnp.float32),
        scratch_types=[
            pltpu.VMEM((L0,), jnp.int32),
            pltpu.VMEM((L1,), jnp.float32),
            pltpu.VMEM((L1,), jnp.int32),
            pltpu.VMEM((L2,), jnp.float32),
            pltpu.VMEM((L2,), jnp.int32),
            pltpu.VMEM((L3,), jnp.float32),
            pltpu.VMEM((ROWS_PER_W * 8,), jnp.int32),
            pltpu.VMEM((ROWS_PER_W * 8,), jnp.float32),
            pltpu.VMEM((GROUP * 8, E), jnp.float32),
            pltpu.VMEM((GROUP * 8, E), jnp.float32),
            pltpu.VMEM((GROUP, E), jnp.float32),
            pltpu.SemaphoreType.DMA,
            pltpu.SemaphoreType.DMA,
        ],
    )
    def k(table_hbm, p0_hbm, w0_hbm, p1_hbm, w1_hbm, p2_hbm, w2_hbm, out_hbm,
          p0_v, w0_v, p1_v, w1_v, p2_v, w2_v, idx_v, wt_v, rows_a, rows_b,
          ob_v, sem_a, sem_b):
        wid = lax.axis_index("s") * NC + lax.axis_index("c")
        b = wid // (NW // B)
        q = wid % (NW // B)
        pltpu.sync_copy(p0_hbm.at[b], p0_v)
        pltpu.sync_copy(w0_hbm.at[b], w0_v)
        pltpu.sync_copy(p1_hbm.at[b], p1_v)
        pltpu.sync_copy(w1_hbm.at[b], w1_v)
        pltpu.sync_copy(p2_hbm.at[b], p2_v)
        pltpu.sync_copy(w2_hbm.at[b], w2_v)
        iota = lax.iota(jnp.int32, LANES)
        bufs = ((rows_a, sem_a), (rows_b, sem_b))

        def compose(g, carry):
            jv = q * ROWS_PER_W + g * GROUP + iota
            a2 = plsc.load_gather(p2_v, [jv])
            b2 = plsc.load_gather(p2_v, [(L2 - 1) - jv])
            w2v = plsc.load_gather(w2_v, [jv])
            lvl1 = []
            for p, w in ((a2, w2v), (b2, 1.0 - w2v)):
                pa = plsc.load_gather(p1_v, [p])
                pb = plsc.load_gather(p1_v, [(L1 - 1) - p])
                w1v = plsc.load_gather(w1_v, [p])
                lvl1.append((pa, w * w1v))
                lvl1.append((pb, w * (1.0 - w1v)))
            kk = 0
            base = g * (GROUP * 8)
            for p, w in lvl1:
                ia = plsc.load_gather(p0_v, [p])
                ib = plsc.load_gather(p0_v, [(L0 - 1) - p])
                w0v = plsc.load_gather(w0_v, [p])
                for idx, wt in ((ia, w * w0v), (ib, w * (1.0 - w0v))):
                    pos = base + iota * 8 + kk
                    plsc.store_scatter(idx_v, [pos], idx + b * L0)
                    plsc.store_scatter(wt_v, [pos], wt)
                    kk += 1
            return carry

        lax.fori_loop(0, NGROUPS, compose, 0)

        def gather_of(g, rows_v, sem):
            return pltpu.make_async_copy(
                table_hbm.at[idx_v.at[pl.ds(g * (GROUP * 8), GROUP * 8)]],
                rows_v, sem)

        gather_of(0, rows_a, sem_a).start()
        gather_of(1, rows_b, sem_b).start()

        def outer(i, carry):
            for slot in range(2):
                g = i * 2 + slot
                rows_v, sem = bufs[slot]
                gather_of(g, rows_v, sem).wait()
                wbase = g * (GROUP * 8)

                def inner(jj, c2):
                    accs = [jnp.zeros((LANES,), jnp.float32)] * 8
                    for t in range(8):
                        rs = _splat(jj * 8 + t)
                        wv = plsc.load_gather(wt_v, [wbase + rs])
                        for d in range(8):
                            accs[d] = accs[d] + wv * plsc.load_gather(
                                rows_v, [rs, d * LANES + iota])
                    js = _splat(jj)
                    for d in range(8):
                        plsc.store_scatter(ob_v, [js, d * LANES + iota],
                                           accs[d])
                    return c2

                lax.fori_loop(0, GROUP, inner, 0)
                row0 = b * L3 + q * ROWS_PER_W + g * GROUP
                pltpu.sync_copy(ob_v, out_hbm.at[pl.ds(row0, GROUP)])

                @pl.when(g + 2 < NGROUPS)
                def _():
                    gather_of(g + 2, rows_v, sem).start()
            return carry

        lax.fori_loop(0, NGROUPS // 2, outer, 0)

    return k(table, p0, w0, p1, w1, p2, w2)


def kernel(embs, scores):
    s = scores[..., 0]
    p0, w0, s1 = _tc_layer(s)
    p1, w1, p2, w2, s3 = _tc_layers12(s1)
    table = embs.reshape(B * L0, E)
    out = _sc_combine(table, p0, w0, p1, w1, p2, w2)
    return out.reshape(B, L3, E), s3
